# baseline (speedup 1.0000x reference)
import functools

import jax
import jax.numpy as jnp
from jax import lax
from jax.experimental import pallas as pl
from jax.experimental.pallas import tpu as pltpu
from jax.experimental.pallas import tpu_sc as plsc

_IMIN = -2147483648


def _score_block(x_ref, s_ref, p_ref):
    xb = x_ref[...]
    s = s_ref[...]
    n2 = jnp.sum(xb * xb, axis=1, keepdims=True)
    inv = jax.lax.rsqrt(jnp.maximum(n2, 1e-24))
    y = jax.lax.dot_general(
        s, xb * inv, (((1,), (1,)), ((), ())),
        preferred_element_type=jnp.float32)
    bits = jax.lax.bitcast_convert_type(y, jnp.int32)
    key = bits ^ ((bits >> 31) & 0x7FFFFFFF)
    rank = 7 - jax.lax.broadcasted_iota(jnp.int32, y.shape, 0)
    p_ref[...] = (key & ~7) | rank


def _tc_scores(x, phase_signatures):
    n, d = x.shape
    e = phase_signatures.shape[0]
    block = 4096
    return pl.pallas_call(
        _score_block,
        grid=(n // block,),
        in_specs=[
            pl.BlockSpec((block, d), lambda b: (b, 0)),
            pl.BlockSpec((e, d), lambda b: (0, 0)),
        ],
        out_specs=pl.BlockSpec((e, block), lambda b: (0, b)),
        out_shape=jax.ShapeDtypeStruct((e, n), jnp.int32),
    )(x, phase_signatures)


def _make_sc_top2(n):
    info = plsc.get_sparse_core_info()
    nw = info.num_cores * info.num_subcores
    per = n // nw
    mesh = plsc.VectorSubcoreMesh(core_axis_name="c", subcore_axis_name="s")

    @functools.partial(
        pl.kernel, mesh=mesh,
        out_type=[
            jax.ShapeDtypeStruct((2, n), jnp.float32),
            jax.ShapeDtypeStruct((2, n), jnp.int32),
        ],
        scratch_types=[
            pltpu.VMEM((8, per), jnp.int32),
            pltpu.VMEM((2, per), jnp.float32),
            pltpu.VMEM((2, per), jnp.int32),
        ],
    )
    def sc_top2(p_hbm, w_hbm, i_hbm, p_v, w_v, i_v):
        wid = lax.axis_index("s") * info.num_cores + lax.axis_index("c")
        base = wid * per
        pltpu.sync_copy(p_hbm.at[:, pl.ds(base, per)], p_v)

        def chunk(j, carry):
            o = j * 16
            v0 = p_v[0, pl.ds(o, 16)]
            v1 = p_v[1, pl.ds(o, 16)]
            v2 = p_v[2, pl.ds(o, 16)]
            v3 = p_v[3, pl.ds(o, 16)]
            v4 = p_v[4, pl.ds(o, 16)]
            v5 = p_v[5, pl.ds(o, 16)]
            v6 = p_v[6, pl.ds(o, 16)]
            v7 = p_v[7, pl.ds(o, 16)]
            a = jnp.maximum(jnp.maximum(v0, v1), jnp.maximum(v2, v3))
            b = jnp.maximum(jnp.maximum(v4, v5), jnp.maximum(v6, v7))
            p1 = jnp.maximum(a, b)
            neg = jnp.full((16,), _IMIN, jnp.int32)
            p2 = neg
            for v in (v0, v1, v2, v3, v4, v5, v6, v7):
                p2 = jnp.maximum(p2, jnp.where(v == p1, neg, v))

            for row, pv in ((0, p1), (1, p2)):
                i_v[row, pl.ds(o, 16)] = 7 - (pv & 7)
                vb = pv & ~7
                w = jax.lax.bitcast_convert_type(
                    vb ^ ((vb >> 31) & 0x7FFFFFFF), jnp.float32)
                w_v[row, pl.ds(o, 16)] = jnp.maximum(w, 0.0)
            return carry

        lax.fori_loop(0, per // 16, chunk, 0)
        pltpu.sync_copy(w_v, w_hbm.at[:, pl.ds(base, per)])
        pltpu.sync_copy(i_v, i_hbm.at[:, pl.ds(base, per)])

    return sc_top2


@functools.partial(jax.jit, static_argnames=())
def kernel(x, phase_signatures):
    n = x.shape[0]
    packed = _tc_scores(x, phase_signatures)
    w_t, i_t = _make_sc_top2(n)(packed)
    return (w_t.T, i_t.T)


# speedup vs baseline: 1.6040x; 1.6040x over previous
import functools

import jax
import jax.numpy as jnp
from jax.experimental import pallas as pl

_IMIN = -2147483648


def _router_block(x_ref, s_ref, w_ref, i_ref):
    xb = x_ref[...]
    s = s_ref[...]
    n2 = jnp.sum(xb * xb, axis=1, keepdims=True)
    inv = jax.lax.rsqrt(jnp.maximum(n2, 1e-24))
    xn = xb * inv
    y = jax.lax.dot_general(
        s, xn, (((1,), (1,)), ((), ())),
        preferred_element_type=jnp.float32)

    bits = jax.lax.bitcast_convert_type(y, jnp.int32)
    key = bits ^ ((bits >> 31) & 0x7FFFFFFF)
    rank = 7 - jax.lax.broadcasted_iota(jnp.int32, y.shape, 0)
    packed = (key & ~7) | rank

    p1 = jnp.max(packed, axis=0, keepdims=True)
    p2 = jnp.max(jnp.where(packed == p1, _IMIN, packed), axis=0, keepdims=True)

    pv = jnp.concatenate([p1, p2], axis=0)
    i_ref[...] = 7 - (pv & 7)
    vbits = pv & ~7
    w = jax.lax.bitcast_convert_type(
        vbits ^ ((vbits >> 31) & 0x7FFFFFFF), jnp.float32)
    w_ref[...] = jnp.maximum(w, 0.0)


@functools.partial(jax.jit, static_argnames=())
def kernel(x, phase_signatures):
    n, d = x.shape
    e = phase_signatures.shape[0]
    block = 4096
    grid = (n // block,)
    w_t, i_t = pl.pallas_call(
        _router_block,
        grid=grid,
        in_specs=[
            pl.BlockSpec((block, d), lambda b: (b, 0)),
            pl.BlockSpec((e, d), lambda b: (0, 0)),
        ],
        out_specs=[
            pl.BlockSpec((2, block), lambda b: (0, b)),
            pl.BlockSpec((2, block), lambda b: (0, b)),
        ],
        out_shape=[
            jax.ShapeDtypeStruct((2, n), jnp.float32),
            jax.ShapeDtypeStruct((2, n), jnp.int32),
        ],
    )(x, phase_signatures)
    return (w_t.T, i_t.T)
